# baseline (speedup 1.0000x reference)
import functools

import jax
import jax.numpy as jnp
from jax import lax
from jax.experimental import pallas as pl
from jax.experimental.pallas import tpu as pltpu
from jax.experimental.pallas import tpu_sc as plsc

S, H = 2048, 1024
E, I_DIM, IS_DIM, TOPK = 64, 512, 512, 2
T = 64
N_TILES = 128
P_MAX = N_TILES * T
TOKEN_BLK = 256
N_TOK_BLKS = S // TOKEN_BLK


def _router_shared_body(x_ref, wgate_ref, bias_ref, wg_ref, wu_ref, wd_ref,
                        shared_ref, sel0_ref, sel1_ref, w0_ref, w1_ref,
                        x16_ref):
    xb = x_ref[...]
    logits = lax.dot_general(xb, wgate_ref[...], (((1,), (1,)), ((), ())),
                             preferred_element_type=jnp.float32)
    scores = jax.nn.sigmoid(logits)
    biased = scores + bias_ref[...]
    col = lax.broadcasted_iota(jnp.int32, biased.shape, 1)
    m1 = jnp.max(biased, axis=1, keepdims=True)
    a1 = jnp.min(jnp.where(biased == m1, col, E), axis=1, keepdims=True)
    b2 = jnp.where(col == a1, -jnp.inf, biased)
    m2 = jnp.max(b2, axis=1, keepdims=True)
    a2 = jnp.min(jnp.where(b2 == m2, col, E), axis=1, keepdims=True)
    s1 = jnp.sum(jnp.where(col == a1, scores, 0.0), axis=1)
    s2 = jnp.sum(jnp.where(col == a2, scores, 0.0), axis=1)
    den = s1 + s2 + 1e-20
    sel0_ref[0, 0, :] = a1[:, 0]
    sel1_ref[0, 0, :] = a2[:, 0]
    w0_ref[0, 0, :] = s1 / den
    w1_ref[0, 0, :] = s2 / den
    xb16 = xb.astype(jnp.bfloat16)
    x16_ref[...] = xb16
    wg = wg_ref[...].astype(jnp.bfloat16)
    wu = wu_ref[...].astype(jnp.bfloat16)
    wd = wd_ref[...].astype(jnp.bfloat16)
    g = lax.dot_general(xb16, wg, (((1,), (1,)), ((), ())),
                        preferred_element_type=jnp.float32)
    u = lax.dot_general(xb16, wu, (((1,), (1,)), ((), ())),
                        preferred_element_type=jnp.float32)
    hmid = (g * jax.nn.sigmoid(g) * u).astype(jnp.bfloat16)
    shared_ref[...] = lax.dot_general(hmid, wd, (((1,), (1,)), ((), ())),
                                      preferred_element_type=jnp.float32)


def _router_shared(tokens, W_gate, bias2d, Wg, Wu, Wd):
    return pl.pallas_call(
        _router_shared_body,
        grid=(N_TOK_BLKS,),
        in_specs=[
            pl.BlockSpec((TOKEN_BLK, H), lambda i: (i, 0)),
            pl.BlockSpec((E, H), lambda i: (0, 0)),
            pl.BlockSpec((1, E), lambda i: (0, 0)),
            pl.BlockSpec((IS_DIM, H), lambda i: (0, 0)),
            pl.BlockSpec((IS_DIM, H), lambda i: (0, 0)),
            pl.BlockSpec((H, IS_DIM), lambda i: (0, 0)),
        ],
        out_specs=[
            pl.BlockSpec((TOKEN_BLK, H), lambda i: (i, 0)),
            pl.BlockSpec((1, 1, TOKEN_BLK), lambda i: (i, 0, 0)),
            pl.BlockSpec((1, 1, TOKEN_BLK), lambda i: (i, 0, 0)),
            pl.BlockSpec((1, 1, TOKEN_BLK), lambda i: (i, 0, 0)),
            pl.BlockSpec((1, 1, TOKEN_BLK), lambda i: (i, 0, 0)),
            pl.BlockSpec((TOKEN_BLK, H), lambda i: (i, 0)),
        ],
        out_shape=[
            jax.ShapeDtypeStruct((S, H), jnp.float32),
            jax.ShapeDtypeStruct((N_TOK_BLKS, 1, TOKEN_BLK), jnp.int32),
            jax.ShapeDtypeStruct((N_TOK_BLKS, 1, TOKEN_BLK), jnp.int32),
            jax.ShapeDtypeStruct((N_TOK_BLKS, 1, TOKEN_BLK), jnp.float32),
            jax.ShapeDtypeStruct((N_TOK_BLKS, 1, TOKEN_BLK), jnp.float32),
            jax.ShapeDtypeStruct((S, H), jnp.bfloat16),
        ],
    )(tokens, W_gate, bias2d, Wg, Wu, Wd)


def _group_body(te_ref, xr_ref, eg_ref, eu_ref, ed_ref, wp_ref, y_ref):
    xb = xr_ref[...]
    eg = eg_ref[0].astype(jnp.bfloat16)
    eu = eu_ref[0].astype(jnp.bfloat16)
    g = lax.dot_general(xb, eg, (((1,), (0,)), ((), ())),
                        preferred_element_type=jnp.float32)
    u = lax.dot_general(xb, eu, (((1,), (0,)), ((), ())),
                        preferred_element_type=jnp.float32)
    hmid = (g * jax.nn.sigmoid(g) * u).astype(jnp.bfloat16)
    ed = ed_ref[0].astype(jnp.bfloat16)
    y = lax.dot_general(hmid, ed, (((1,), (0,)), ((), ())),
                        preferred_element_type=jnp.float32)
    y_ref[...] = y * wp_ref[0]


def _grouped_mlp(tile_expert, x_rows, Eg, Eu, Ed, w_pad):
    grid_spec = pltpu.PrefetchScalarGridSpec(
        num_scalar_prefetch=1,
        grid=(N_TILES,),
        in_specs=[
            pl.BlockSpec((T, H), lambda i, te: (i, 0)),
            pl.BlockSpec((1, H, I_DIM), lambda i, te: (te[i], 0, 0)),
            pl.BlockSpec((1, H, I_DIM), lambda i, te: (te[i], 0, 0)),
            pl.BlockSpec((1, I_DIM, H), lambda i, te: (te[i], 0, 0)),
            pl.BlockSpec((1, T, 1), lambda i, te: (i, 0, 0)),
        ],
        out_specs=pl.BlockSpec((T, H), lambda i, te: (i, 0)),
    )
    return pl.pallas_call(
        _group_body,
        grid_spec=grid_spec,
        out_shape=jax.ShapeDtypeStruct((P_MAX, H), jnp.float32),
    )(tile_expert, x_rows, Eg, Eu, Ed, w_pad)


NC, NS = 2, 16
NW = NC * NS
GCH = 32
_SC_MESH = dict(core_axis_name="c", subcore_axis_name="s")


def _sc_row_gather(table, idx, chunk):
    dt = table.dtype
    ncol = table.shape[1]
    n = idx.shape[0]
    per_w = n // NW
    nch = per_w // chunk
    idx3 = idx.reshape(NW, nch, chunk)

    @functools.partial(
        pl.kernel,
        mesh=plsc.VectorSubcoreMesh(**_SC_MESH),
        out_type=jax.ShapeDtypeStruct((n, ncol), dt),
        scratch_types=[
            pltpu.VMEM((nch, chunk), jnp.int32),
            pltpu.VMEM((chunk, ncol), dt),
            pltpu.VMEM((chunk, ncol), dt),
            pltpu.SemaphoreType.DMA,
            pltpu.SemaphoreType.DMA,
        ],
    )
    def k(tab_hbm, idx_hbm, out_hbm, idx_v, buf0, buf1, sem0, sem1):
        wid = lax.axis_index("s") * NC + lax.axis_index("c")
        pltpu.sync_copy(idx_hbm.at[wid], idx_v)
        bufs = (buf0, buf1)
        sems = (sem0, sem1)
        cps = [pltpu.async_copy(tab_hbm.at[idx_v.at[c]], bufs[c], sems[c])
               for c in range(min(2, nch))]
        for c in range(nch):
            b = c % 2
            cps[b].wait()
            pltpu.sync_copy(
                bufs[b], out_hbm.at[pl.ds((wid * nch + c) * chunk, chunk)])
            if c + 2 < nch:
                cps[b] = pltpu.async_copy(
                    tab_hbm.at[idx_v.at[c + 2]], bufs[b], sems[b])

    return k(table, idx3)


def _add3_body(a_ref, b_ref, c_ref, o_ref):
    o_ref[...] = a_ref[...] + b_ref[...] + c_ref[...]


def _tc_add3(a, b, c):
    return pl.pallas_call(
        _add3_body,
        grid=(N_TOK_BLKS,),
        in_specs=[pl.BlockSpec((TOKEN_BLK, H), lambda i: (i, 0))] * 3,
        out_specs=pl.BlockSpec((TOKEN_BLK, H), lambda i: (i, 0)),
        out_shape=jax.ShapeDtypeStruct((S, H), jnp.float32),
    )(a, b, c)


def kernel(x, W_gate, expert_bias, Wg, Wu, Wd, Eg, Eu, Ed):
    tokens = x.reshape(S, H)
    shared, sel0, sel1, w0, w1, x16 = _router_shared(
        tokens, W_gate, expert_bias.reshape(1, E), Wg, Wu, Wd)
    sel0 = sel0.reshape(S)
    sel1 = sel1.reshape(S)
    w0 = w0.reshape(S)
    w1 = w1.reshape(S)

    flat_e = jnp.stack([sel0, sel1], axis=1).reshape(-1)
    flat_w = jnp.stack([w0, w1], axis=1).reshape(-1)
    flat_t = jnp.arange(S * TOPK, dtype=jnp.int32) // TOPK
    onehot = (flat_e[:, None] == jnp.arange(E, dtype=jnp.int32)[None, :])
    cum = jnp.cumsum(onehot.astype(jnp.int32), axis=0)
    rank = jnp.take_along_axis(cum, flat_e[:, None], axis=1)[:, 0] - 1
    counts = cum[-1]
    tiles_per_e = (counts + T - 1) // T
    incl = jnp.cumsum(tiles_per_e)
    tile_off = incl - tiles_per_e
    used = incl[-1]
    pad_pos = (tile_off[flat_e] * T + rank).astype(jnp.int32)
    tile_ids = jnp.arange(N_TILES, dtype=jnp.int32)
    ss = jnp.minimum(jnp.searchsorted(incl, tile_ids, side='right'),
                     E - 1).astype(jnp.int32)
    last_e = jnp.minimum(jnp.searchsorted(incl, used - 1, side='right'),
                         E - 1).astype(jnp.int32)
    tile_expert = jnp.where(tile_ids < used, ss, last_e)
    row_src = jnp.zeros((P_MAX,), jnp.int32).at[pad_pos].set(flat_t)
    w_pad = jnp.zeros((P_MAX,), jnp.float32).at[pad_pos].set(flat_w)
    w_pad = w_pad.reshape(N_TILES, T, 1)
    inv = pad_pos.reshape(S, TOPK)

    x16i = lax.bitcast_convert_type(
        x16.reshape(S, H // 2, 2), jnp.int32)
    xr_i = _sc_row_gather(x16i, row_src, 2 * GCH)
    x_rows = lax.bitcast_convert_type(
        xr_i, jnp.bfloat16).reshape(P_MAX, H)

    y_pad = _grouped_mlp(tile_expert, x_rows, Eg, Eu, Ed, w_pad)

    y2 = _sc_row_gather(
        y_pad, jnp.concatenate([inv[:, 0], inv[:, 1]]), GCH)
    out = _tc_add3(shared, y2[:S], y2[S:])
    return out.reshape(1, S, H)


# speedup vs baseline: 1.7700x; 1.7700x over previous
import functools

import jax
import jax.numpy as jnp
from jax import lax
from jax.experimental import pallas as pl
from jax.experimental.pallas import tpu as pltpu
from jax.experimental.pallas import tpu_sc as plsc

S, H = 2048, 1024
E, I_DIM, IS_DIM, TOPK = 64, 512, 512, 2
T = 64
N_TILES = 128
P_MAX = N_TILES * T
TOKEN_BLK = 256
N_TOK_BLKS = S // TOKEN_BLK


def _router_shared_body(x_ref, wgate_ref, bias_ref, wg_ref, wu_ref, wd_ref,
                        shared_ref, sel0_ref, sel1_ref, w0_ref, w1_ref):
    xb = x_ref[...]
    logits = lax.dot_general(xb, wgate_ref[...], (((1,), (1,)), ((), ())),
                             preferred_element_type=jnp.float32)
    scores = jax.nn.sigmoid(logits)
    biased = scores + bias_ref[...]
    col = lax.broadcasted_iota(jnp.int32, biased.shape, 1)
    m1 = jnp.max(biased, axis=1, keepdims=True)
    a1 = jnp.min(jnp.where(biased == m1, col, E), axis=1, keepdims=True)
    b2 = jnp.where(col == a1, -jnp.inf, biased)
    m2 = jnp.max(b2, axis=1, keepdims=True)
    a2 = jnp.min(jnp.where(b2 == m2, col, E), axis=1, keepdims=True)
    s1 = jnp.sum(jnp.where(col == a1, scores, 0.0), axis=1)
    s2 = jnp.sum(jnp.where(col == a2, scores, 0.0), axis=1)
    den = s1 + s2 + 1e-20
    sel0_ref[0, 0, :] = a1[:, 0]
    sel1_ref[0, 0, :] = a2[:, 0]
    w0_ref[0, 0, :] = s1 / den
    w1_ref[0, 0, :] = s2 / den
    xb16 = xb.astype(jnp.bfloat16)
    wg = wg_ref[...].astype(jnp.bfloat16)
    wu = wu_ref[...].astype(jnp.bfloat16)
    wd = wd_ref[...].astype(jnp.bfloat16)
    g = lax.dot_general(xb16, wg, (((1,), (1,)), ((), ())),
                        preferred_element_type=jnp.float32)
    u = lax.dot_general(xb16, wu, (((1,), (1,)), ((), ())),
                        preferred_element_type=jnp.float32)
    hmid = (g * jax.nn.sigmoid(g) * u).astype(jnp.bfloat16)
    shared_ref[...] = lax.dot_general(hmid, wd, (((1,), (1,)), ((), ())),
                                      preferred_element_type=jnp.float32)


def _router_shared(tokens, W_gate, bias2d, Wg, Wu, Wd):
    return pl.pallas_call(
        _router_shared_body,
        grid=(N_TOK_BLKS,),
        in_specs=[
            pl.BlockSpec((TOKEN_BLK, H), lambda i: (i, 0)),
            pl.BlockSpec((E, H), lambda i: (0, 0)),
            pl.BlockSpec((1, E), lambda i: (0, 0)),
            pl.BlockSpec((IS_DIM, H), lambda i: (0, 0)),
            pl.BlockSpec((IS_DIM, H), lambda i: (0, 0)),
            pl.BlockSpec((H, IS_DIM), lambda i: (0, 0)),
        ],
        out_specs=[
            pl.BlockSpec((TOKEN_BLK, H), lambda i: (i, 0)),
            pl.BlockSpec((1, 1, TOKEN_BLK), lambda i: (i, 0, 0)),
            pl.BlockSpec((1, 1, TOKEN_BLK), lambda i: (i, 0, 0)),
            pl.BlockSpec((1, 1, TOKEN_BLK), lambda i: (i, 0, 0)),
            pl.BlockSpec((1, 1, TOKEN_BLK), lambda i: (i, 0, 0)),
        ],
        out_shape=[
            jax.ShapeDtypeStruct((S, H), jnp.float32),
            jax.ShapeDtypeStruct((N_TOK_BLKS, 1, TOKEN_BLK), jnp.int32),
            jax.ShapeDtypeStruct((N_TOK_BLKS, 1, TOKEN_BLK), jnp.int32),
            jax.ShapeDtypeStruct((N_TOK_BLKS, 1, TOKEN_BLK), jnp.float32),
            jax.ShapeDtypeStruct((N_TOK_BLKS, 1, TOKEN_BLK), jnp.float32),
        ],
    )(tokens, W_gate, bias2d, Wg, Wu, Wd)


def _group_body(te_ref, xr_ref, eg_ref, eu_ref, ed_ref, wp_ref, y_ref):
    xb = xr_ref[...].astype(jnp.bfloat16)
    eg = eg_ref[0].astype(jnp.bfloat16)
    eu = eu_ref[0].astype(jnp.bfloat16)
    g = lax.dot_general(xb, eg, (((1,), (0,)), ((), ())),
                        preferred_element_type=jnp.float32)
    u = lax.dot_general(xb, eu, (((1,), (0,)), ((), ())),
                        preferred_element_type=jnp.float32)
    hmid = (g * jax.nn.sigmoid(g) * u).astype(jnp.bfloat16)
    ed = ed_ref[0].astype(jnp.bfloat16)
    y = lax.dot_general(hmid, ed, (((1,), (0,)), ((), ())),
                        preferred_element_type=jnp.float32)
    y_ref[...] = y * wp_ref[0]


def _grouped_mlp(tile_expert, x_rows, Eg, Eu, Ed, w_pad):
    grid_spec = pltpu.PrefetchScalarGridSpec(
        num_scalar_prefetch=1,
        grid=(N_TILES,),
        in_specs=[
            pl.BlockSpec((T, H), lambda i, te: (i, 0)),
            pl.BlockSpec((1, H, I_DIM), lambda i, te: (te[i], 0, 0)),
            pl.BlockSpec((1, H, I_DIM), lambda i, te: (te[i], 0, 0)),
            pl.BlockSpec((1, I_DIM, H), lambda i, te: (te[i], 0, 0)),
            pl.BlockSpec((1, T, 1), lambda i, te: (i, 0, 0)),
        ],
        out_specs=pl.BlockSpec((T, H), lambda i, te: (i, 0)),
    )
    return pl.pallas_call(
        _group_body,
        grid_spec=grid_spec,
        out_shape=jax.ShapeDtypeStruct((P_MAX, H), jnp.float32),
    )(tile_expert, x_rows, Eg, Eu, Ed, w_pad)


NC, NS = 2, 16
NW = NC * NS
GCH = 32
_SC_MESH = dict(core_axis_name="c", subcore_axis_name="s")


def _sc_row_gather(table, idx, chunk):
    dt = table.dtype
    ncol = table.shape[1]
    n = idx.shape[0]
    per_w = n // NW
    nch = per_w // chunk
    idx3 = idx.reshape(NW, nch, chunk)

    @functools.partial(
        pl.kernel,
        mesh=plsc.VectorSubcoreMesh(**_SC_MESH),
        out_type=jax.ShapeDtypeStruct((n, ncol), dt),
        scratch_types=[
            pltpu.VMEM((nch, chunk), jnp.int32),
            pltpu.VMEM((chunk, ncol), dt),
            pltpu.VMEM((chunk, ncol), dt),
            pltpu.SemaphoreType.DMA,
            pltpu.SemaphoreType.DMA,
        ],
    )
    def k(tab_hbm, idx_hbm, out_hbm, idx_v, buf0, buf1, sem0, sem1):
        wid = lax.axis_index("s") * NC + lax.axis_index("c")
        pltpu.sync_copy(idx_hbm.at[wid], idx_v)
        bufs = (buf0, buf1)
        sems = (sem0, sem1)
        cps = [pltpu.async_copy(tab_hbm.at[idx_v.at[c]], bufs[c], sems[c])
               for c in range(min(2, nch))]
        for c in range(nch):
            b = c % 2
            cps[b].wait()
            pltpu.sync_copy(
                bufs[b], out_hbm.at[pl.ds((wid * nch + c) * chunk, chunk)])
            if c + 2 < nch:
                cps[b] = pltpu.async_copy(
                    tab_hbm.at[idx_v.at[c + 2]], bufs[b], sems[b])

    return k(table, idx3)


def _add3_body(a_ref, b_ref, c_ref, o_ref):
    o_ref[...] = a_ref[...] + b_ref[...] + c_ref[...]


def _tc_add3(a, b, c):
    return pl.pallas_call(
        _add3_body,
        grid=(N_TOK_BLKS,),
        in_specs=[pl.BlockSpec((TOKEN_BLK, H), lambda i: (i, 0))] * 3,
        out_specs=pl.BlockSpec((TOKEN_BLK, H), lambda i: (i, 0)),
        out_shape=jax.ShapeDtypeStruct((S, H), jnp.float32),
    )(a, b, c)


def kernel(x, W_gate, expert_bias, Wg, Wu, Wd, Eg, Eu, Ed):
    tokens = x.reshape(S, H)
    shared, sel0, sel1, w0, w1 = _router_shared(
        tokens, W_gate, expert_bias.reshape(1, E), Wg, Wu, Wd)
    sel0 = sel0.reshape(S)
    sel1 = sel1.reshape(S)
    w0 = w0.reshape(S)
    w1 = w1.reshape(S)

    flat_e = jnp.stack([sel0, sel1], axis=1).reshape(-1)
    flat_w = jnp.stack([w0, w1], axis=1).reshape(-1)
    flat_t = jnp.arange(S * TOPK, dtype=jnp.int32) // TOPK
    onehot = (flat_e[:, None] == jnp.arange(E, dtype=jnp.int32)[None, :])
    cum = jnp.cumsum(onehot.astype(jnp.int32), axis=0)
    rank = jnp.take_along_axis(cum, flat_e[:, None], axis=1)[:, 0] - 1
    counts = cum[-1]
    tiles_per_e = (counts + T - 1) // T
    incl = jnp.cumsum(tiles_per_e)
    tile_off = incl - tiles_per_e
    used = incl[-1]
    pad_pos = (tile_off[flat_e] * T + rank).astype(jnp.int32)
    tile_ids = jnp.arange(N_TILES, dtype=jnp.int32)
    ss = jnp.minimum(jnp.searchsorted(incl, tile_ids, side='right'),
                     E - 1).astype(jnp.int32)
    last_e = jnp.minimum(jnp.searchsorted(incl, used - 1, side='right'),
                         E - 1).astype(jnp.int32)
    tile_expert = jnp.where(tile_ids < used, ss, last_e)
    row_src = (jnp.arange(P_MAX, dtype=jnp.int32) % S).at[pad_pos].set(
        flat_t)
    w_pad = jnp.zeros((P_MAX,), jnp.float32).at[pad_pos].set(flat_w)
    w_pad = w_pad.reshape(N_TILES, T, 1)
    inv = pad_pos.reshape(S, TOPK)

    x_rows = _sc_row_gather(tokens, row_src, GCH)

    y_pad = _grouped_mlp(tile_expert, x_rows, Eg, Eu, Ed, w_pad)

    y2 = _sc_row_gather(
        y_pad, jnp.concatenate([inv[:, 0], inv[:, 1]]), GCH)
    out = _tc_add3(shared, y2[:S], y2[S:])
    return out.reshape(1, S, H)
